# Initial kernel scaffold; baseline (speedup 1.0000x reference)
import jax
import jax.numpy as jnp
from jax.experimental import pallas as pl


def kernel(agent_feat, gw1, gb1, gw2, gb2, ew1, eb1, ew2, eb2, aw1, ab1, aw2, ab2):
    raise NotImplementedError("write your pallas kernel here")


# speedup vs baseline: 4.8547x; 4.8547x over previous
import functools

import jax
import jax.numpy as jnp
from jax.experimental import pallas as pl


def _gelu(x):
    return 0.5 * x * (1.0 + jax.lax.erf(x * (2.0 ** -0.5)))


def _body(x_ref, gw1_ref, gb1_ref, gw2_ref, gb2_ref,
          ew1_ref, eb1_ref, ew2_ref, eb2_ref,
          aw1_ref, ab1_ref, aw2_ref, ab2_ref,
          logits_ref, alpha_ref, gates_ref, load_ref, *, E):
    x = x_ref[...]

    g = _gelu(jnp.dot(x, gw1_ref[...], preferred_element_type=jnp.float32)
              + gb1_ref[...])
    gl = jnp.dot(g, gw2_ref[...], preferred_element_type=jnp.float32) + gb2_ref[...]

    ids = jax.lax.broadcasted_iota(jnp.int32, gl.shape, 1)
    m1 = jnp.max(gl, axis=-1, keepdims=True)
    i1 = jnp.min(jnp.where(gl == m1, ids, E), axis=-1, keepdims=True)
    masked = jnp.where(ids == i1, -jnp.inf, gl)
    m2 = jnp.max(masked, axis=-1, keepdims=True)
    i2 = jnp.min(jnp.where(masked == m2, ids, E), axis=-1, keepdims=True)
    keep = (ids == i1) | (ids == i2)
    sparse = jnp.where(keep, gl, 0.0)
    mx = jnp.maximum(m1, 0.0)
    ex = jnp.exp(sparse - mx)
    gwts = ex / jnp.sum(ex, axis=-1, keepdims=True)
    gates_ref[...] = gwts

    @pl.when(pl.program_id(0) == 0)
    def _():
        load_ref[...] = jnp.zeros_like(load_ref)

    load_ref[...] += jnp.sum(gwts, axis=0, keepdims=True)

    acc = jnp.dot(gwts, eb2_ref[...], preferred_element_type=jnp.float32)
    for e in range(E):
        h = _gelu(jnp.dot(x, ew1_ref[e], preferred_element_type=jnp.float32)
                  + eb1_ref[e][None, :])
        acc += gwts[:, e:e + 1] * jnp.dot(h, ew2_ref[e],
                                          preferred_element_type=jnp.float32)
    logits_ref[...] = acc

    ah = _gelu(jnp.dot(x, aw1_ref[...], preferred_element_type=jnp.float32)
               + ab1_ref[...])
    z = jnp.dot(ah, aw2_ref[...], preferred_element_type=jnp.float32) + ab2_ref[...]
    alpha_ref[...] = jnp.maximum(z, 0.0) + jnp.log1p(jnp.exp(-jnp.abs(z)))


def kernel(agent_feat, gw1, gb1, gw2, gb2, ew1, eb1, ew2, eb2, aw1, ab1, aw2, ab2):
    B, D = agent_feat.shape
    E = gw2.shape[1]
    H = ew1.shape[2]
    C = ew2.shape[2]
    TB = min(512, B)
    nb = B // TB

    full = lambda shape: pl.BlockSpec(shape, lambda i: (0,) * len(shape))
    out = pl.pallas_call(
        functools.partial(_body, E=E),
        grid=(nb,),
        in_specs=[
            pl.BlockSpec((TB, D), lambda i: (i, 0)),
            full((D, D)), full((1, D)), full((D, E)), full((1, E)),
            full((E, D, H)), full((E, H)), full((E, H, C)), full((E, C)),
            full((D, H)), full((1, H)), full((H, C)), full((1, C)),
        ],
        out_specs=[
            pl.BlockSpec((TB, C), lambda i: (i, 0)),
            pl.BlockSpec((TB, C), lambda i: (i, 0)),
            pl.BlockSpec((TB, E), lambda i: (i, 0)),
            pl.BlockSpec((1, E), lambda i: (0, 0)),
        ],
        out_shape=[
            jax.ShapeDtypeStruct((B, C), jnp.float32),
            jax.ShapeDtypeStruct((B, C), jnp.float32),
            jax.ShapeDtypeStruct((B, E), jnp.float32),
            jax.ShapeDtypeStruct((1, E), jnp.float32),
        ],
    )(agent_feat, gw1, gb1.reshape(1, D), gw2, gb2.reshape(1, E),
      ew1, eb1, ew2, eb2,
      aw1, ab1.reshape(1, H), aw2, ab2.reshape(1, C))

    logits, alpha, gate_weights, load = out
    return (logits, alpha, gate_weights, load.reshape(E))


# speedup vs baseline: 5.0097x; 1.0319x over previous
import functools

import jax
import jax.numpy as jnp
from jax.experimental import pallas as pl


def _gelu(x):
    return 0.5 * x * (1.0 + jax.lax.erf(x * (2.0 ** -0.5)))


def _body(x_ref, gw1_ref, gb1_ref, gw2_ref, gb2_ref,
          ew1_ref, eb1_ref, ew2_ref, eb2_ref,
          aw1_ref, ab1_ref, aw2_ref, ab2_ref,
          logits_ref, alpha_ref, gates_ref, load_ref, *, E):
    x = x_ref[...]

    g = _gelu(jnp.dot(x, gw1_ref[...], preferred_element_type=jnp.float32)
              + gb1_ref[...])
    gl = jnp.dot(g, gw2_ref[...], preferred_element_type=jnp.float32) + gb2_ref[...]

    ah = _gelu(jnp.dot(x, aw1_ref[...], preferred_element_type=jnp.float32)
               + ab1_ref[...])

    ids = jax.lax.broadcasted_iota(jnp.int32, gl.shape, 1)
    m1 = jnp.max(gl, axis=-1, keepdims=True)
    i1 = jnp.min(jnp.where(gl == m1, ids, E), axis=-1, keepdims=True)
    masked = jnp.where(ids == i1, -jnp.inf, gl)
    m2 = jnp.max(masked, axis=-1, keepdims=True)
    i2 = jnp.min(jnp.where(masked == m2, ids, E), axis=-1, keepdims=True)
    keep = (ids == i1) | (ids == i2)
    sparse = jnp.where(keep, gl, 0.0)
    mx = jnp.maximum(m1, 0.0)
    ex = jnp.exp(sparse - mx)
    gwts = ex / jnp.sum(ex, axis=-1, keepdims=True)
    gates_ref[...] = gwts

    @pl.when(pl.program_id(0) == 0)
    def _():
        load_ref[...] = jnp.zeros_like(load_ref)

    load_ref[...] += jnp.sum(gwts, axis=0, keepdims=True)

    z = jnp.dot(ah, aw2_ref[...], preferred_element_type=jnp.float32) + ab2_ref[...]
    alpha_ref[...] = jnp.maximum(z, 0.0) + jnp.log1p(jnp.exp(-jnp.abs(z)))

    acc = jnp.dot(gwts, eb2_ref[...], preferred_element_type=jnp.float32)
    for e in range(E):
        h = _gelu(jnp.dot(x, ew1_ref[e], preferred_element_type=jnp.float32)
                  + eb1_ref[e][None, :])
        acc += gwts[:, e:e + 1] * jnp.dot(h, ew2_ref[e],
                                          preferred_element_type=jnp.float32)
    logits_ref[...] = acc


def kernel(agent_feat, gw1, gb1, gw2, gb2, ew1, eb1, ew2, eb2, aw1, ab1, aw2, ab2):
    B, D = agent_feat.shape
    E = gw2.shape[1]
    H = ew1.shape[2]
    C = ew2.shape[2]
    TB = min(512, B)
    nb = B // TB

    full = lambda shape: pl.BlockSpec(shape, lambda i: (0,) * len(shape))
    out = pl.pallas_call(
        functools.partial(_body, E=E),
        grid=(nb,),
        in_specs=[
            pl.BlockSpec((TB, D), lambda i: (i, 0)),
            full((D, D)), full((1, D)), full((D, E)), full((1, E)),
            full((E, D, H)), full((E, H)), full((E, H, C)), full((E, C)),
            full((D, H)), full((1, H)), full((H, C)), full((1, C)),
        ],
        out_specs=[
            pl.BlockSpec((TB, C), lambda i: (i, 0)),
            pl.BlockSpec((TB, C), lambda i: (i, 0)),
            pl.BlockSpec((TB, E), lambda i: (i, 0)),
            pl.BlockSpec((1, E), lambda i: (0, 0)),
        ],
        out_shape=[
            jax.ShapeDtypeStruct((B, C), jnp.float32),
            jax.ShapeDtypeStruct((B, C), jnp.float32),
            jax.ShapeDtypeStruct((B, E), jnp.float32),
            jax.ShapeDtypeStruct((1, E), jnp.float32),
        ],
    )(agent_feat, gw1, gb1.reshape(1, D), gw2, gb2.reshape(1, E),
      ew1, eb1, ew2, eb2,
      aw1, ab1.reshape(1, H), aw2, ab2.reshape(1, C))

    logits, alpha, gate_weights, load = out
    return (logits, alpha, gate_weights, load.reshape(E))
